# baseline (speedup 1.0000x reference)
import functools

import jax
import jax.numpy as jnp
from jax.experimental import pallas as pl
from jax.experimental.pallas import tpu as pltpu

B = 16
LX = 1024
LC = 1024
LT = 2048
D = 64
PAD = -999.0


def _row_kernel(lens_ref, x0_ref, xl_ref, t_ref, o_ref):
    i = pl.program_id(0)
    lx = lens_ref[0, i]
    lc = lens_ref[1, i]

    beta = x0_ref[0, 0, 0]

    r = (lx - 1) % 8
    blk = xl_ref[0]
    row_ids = jax.lax.broadcasted_iota(jnp.int32, (8, D), 0)
    col_ids = jax.lax.broadcasted_iota(jnp.int32, (8, D), 1)
    x_last = jnp.sum(jnp.where((row_ids == r) & (col_ids == 0), blk, 0.0))

    trow = t_ref[0]
    t0 = trow[0, 0]
    t_ids = jax.lax.broadcasted_iota(jnp.int32, (1, LT), 1)
    t_last = jnp.sum(jnp.where(t_ids == lx - 1, trow, 0.0)) - t0
    slope = (x_last - beta) / t_last

    rot = pltpu.roll(trow, LT - lx, 1)
    fut = rot[:, :LC] - t0
    pred = slope * fut + beta

    pos = jax.lax.broadcasted_iota(jnp.int32, (LC, 1), 0)
    col = jnp.where(pos < lc, pred.reshape(LC, 1), PAD)

    d_ids = jax.lax.broadcasted_iota(jnp.int32, (LC, D), 1)
    o_ref[0] = jnp.where(d_ids == 0, col, PAD)


@functools.partial(jax.jit, static_argnames=("interpret",))
def _run(x, time, len_x, len_context, interpret=False):
    lens = jnp.stack([len_x, len_context]).astype(jnp.int32)
    grid_spec = pltpu.PrefetchScalarGridSpec(
        num_scalar_prefetch=1,
        grid=(B,),
        in_specs=[
            pl.BlockSpec((1, 8, D), lambda i, lens: (i, 0, 0)),
            pl.BlockSpec((1, 8, D), lambda i, lens: (i, (lens[0, i] - 1) // 8, 0)),
            pl.BlockSpec((1, 1, LT), lambda i, lens: (i, 0, 0)),
        ],
        out_specs=pl.BlockSpec((1, LC, D), lambda i, lens: (i, 0, 0)),
    )
    return pl.pallas_call(
        _row_kernel,
        grid_spec=grid_spec,
        out_shape=jax.ShapeDtypeStruct((B, LC, D), jnp.float32),
        interpret=interpret,
    )(lens, x, x, time.reshape(B, 1, LT))


def kernel(x, time, context, len_x, len_context):
    return _run(x, time, len_x, len_context)


# speedup vs baseline: 1.1024x; 1.1024x over previous
import functools

import jax
import jax.numpy as jnp
from jax.experimental import pallas as pl
from jax.experimental.pallas import tpu as pltpu

B = 16
LX = 1024
LC = 1024
LT = 2048
D = 64
PAD = -999.0


def _row_kernel(lx_ref, lc_ref, x0_ref, xl_ref, t_ref, o_ref):
    i = pl.program_id(0)
    lx = lx_ref[i]
    lc = lc_ref[i]

    beta = x0_ref[0, 0, 0]

    r = (lx - 1) % 8
    blk = xl_ref[0]
    row_ids = jax.lax.broadcasted_iota(jnp.int32, (8, D), 0)
    col_ids = jax.lax.broadcasted_iota(jnp.int32, (8, D), 1)
    x_last = jnp.sum(jnp.where((row_ids == r) & (col_ids == 0), blk, 0.0))

    trow = t_ref[pl.ds(i, 1)]
    t0 = trow[0, 0]
    t_ids = jax.lax.broadcasted_iota(jnp.int32, (1, LT), 1)
    t_last = jnp.sum(jnp.where(t_ids == lx - 1, trow, 0.0)) - t0
    slope = (x_last - beta) / t_last

    rot = pltpu.roll(trow, LT - lx, 1)
    fut = rot[:, :LC] - t0
    pred = slope * fut + beta

    pos = jax.lax.broadcasted_iota(jnp.int32, (LC, 1), 0)
    col = jnp.where(pos < lc, pred.reshape(LC, 1), PAD)

    d_ids = jax.lax.broadcasted_iota(jnp.int32, (LC, D), 1)
    o_ref[0] = jnp.where(d_ids == 0, col, PAD)


@functools.partial(jax.jit, static_argnames=("interpret",))
def _run(x, time, len_x, len_context, interpret=False):
    grid_spec = pltpu.PrefetchScalarGridSpec(
        num_scalar_prefetch=2,
        grid=(B,),
        in_specs=[
            pl.BlockSpec((1, 8, D), lambda i, lx, lc: (i, 0, 0)),
            pl.BlockSpec((1, 8, D), lambda i, lx, lc: (i, (lx[i] - 1) // 8, 0)),
            pl.BlockSpec((B, LT), lambda i, lx, lc: (0, 0)),
        ],
        out_specs=pl.BlockSpec((1, LC, D), lambda i, lx, lc: (i, 0, 0)),
    )
    return pl.pallas_call(
        _row_kernel,
        grid_spec=grid_spec,
        out_shape=jax.ShapeDtypeStruct((B, LC, D), jnp.float32),
        interpret=interpret,
    )(len_x, len_context, x, x, time)


def kernel(x, time, context, len_x, len_context):
    return _run(x, time, len_x, len_context)
